# baseline (speedup 1.0000x reference)
import functools

import jax
import jax.numpy as jnp
from jax import lax
from jax.experimental import pallas as pl
from jax.experimental.pallas import tpu as pltpu
from jax.experimental.pallas import tpu_sc as plsc

_EPS = 1e-5
_OFFSET_RANGE = 2.0
_SEG = 8
_TC_T = 8192
_CH = 1024
_LANES = 16


def _mlp_seg_kernel(nblk, nvox, x_ref, vb_ref, pts_ref, w1_ref, w2_ref, wf_ref,
                    wh_ref, g1_ref, b1_ref, g2_ref, b2_ref, bf_ref, bh_ref,
                    mean_ref, cnt_ref, loss_ref, pb_ref):
    i = pl.program_id(0)
    pb_ref[...] = pts_ref[0:1, :]
    inv_std = 1.0 / jnp.sqrt(1.0 + _EPS)
    x = x_ref[...]
    h = jnp.dot(w1_ref[...], x, preferred_element_type=jnp.float32)
    h = jnp.maximum(h * (g1_ref[...] * inv_std) + b1_ref[...], 0.0)
    h = jnp.dot(w2_ref[...], h, preferred_element_type=jnp.float32)
    h = jnp.maximum(h * (g2_ref[...] * inv_std) + b2_ref[...], 0.0)
    col = lax.broadcasted_iota(jnp.int32, (1, _TC_T), 1) + i * _TC_T
    valid = col < nvox
    off = jnp.tanh(jnp.dot(wh_ref[...], h, preferred_element_type=jnp.float32)
                   + bh_ref[...]) * _OFFSET_RANGE
    loss_part = jnp.sum(jnp.abs(jnp.where(valid, off, 0.0)))
    aligned = jnp.dot(wf_ref[...], h, preferred_element_type=jnp.float32) + bf_ref[...]
    aligned = jnp.where(valid, aligned, 0.0)
    vb = vb_ref[0:1, :]
    seg_iota = lax.broadcasted_iota(jnp.int32, (_SEG, _TC_T), 0)
    oh = ((vb == seg_iota) & valid).astype(jnp.float32)
    dnums = (((1,), (1,)), ((), ()))
    part_sums = lax.dot_general(aligned, oh, dnums,
                                preferred_element_type=jnp.float32)
    part_cnt = lax.dot_general(jnp.ones((64, _TC_T), jnp.float32), oh, dnums,
                               preferred_element_type=jnp.float32)

    @pl.when(i == 0)
    def _init():
        mean_ref[...] = jnp.zeros_like(mean_ref)
        cnt_ref[...] = jnp.zeros_like(cnt_ref)
        loss_ref[...] = jnp.zeros_like(loss_ref)

    mean_ref[...] += part_sums
    cnt_ref[...] += part_cnt
    loss_ref[...] += loss_part

    @pl.when(i == nblk - 1)
    def _fin():
        c = cnt_ref[...]
        s = mean_ref[...]
        mean_ref[...] = jnp.where(c > 0.0, s / jnp.maximum(c, 1.0), 0.0)
        loss_ref[...] = loss_ref[...] / (nvox * 3.0)


def _run_mlp_seg(xt, vbt, ptst, w1, w2, wf, whp, g1, b1, g2, b2, bf, bhp):
    nvox = xt.shape[1]
    npoint = ptst.shape[1]
    nblk = (nvox + _TC_T - 1) // _TC_T
    pt_t = (npoint // nblk + 127) // 128 * 128
    col = lambda: pl.BlockSpec((64, 1), lambda i: (0, 0))
    wspec = lambda shp: pl.BlockSpec(shp, lambda i: (0, 0))
    return pl.pallas_call(
        functools.partial(_mlp_seg_kernel, nblk, nvox),
        grid=(nblk,),
        in_specs=[
            pl.BlockSpec((64, _TC_T), lambda i: (0, i)),
            pl.BlockSpec((4, _TC_T), lambda i: (0, i)),
            pl.BlockSpec((3, pt_t), lambda i: (0, i)),
            wspec((64, 64)), wspec((64, 64)), wspec((64, 64)), wspec((_SEG, 64)),
            col(), col(), col(), col(), col(),
            wspec((_SEG, 1)),
        ],
        out_specs=[
            pl.BlockSpec((64, _SEG), lambda i: (0, 0)),
            pl.BlockSpec((64, _SEG), lambda i: (0, 0)),
            pl.BlockSpec((_SEG, 128), lambda i: (0, 0)),
            pl.BlockSpec((1, pt_t), lambda i: (0, i)),
        ],
        out_shape=[
            jax.ShapeDtypeStruct((64, _SEG), jnp.float32),
            jax.ShapeDtypeStruct((64, _SEG), jnp.float32),
            jax.ShapeDtypeStruct((_SEG, 128), jnp.float32),
            jax.ShapeDtypeStruct((1, npoint), jnp.int32),
        ],
    )(xt, vbt, ptst, w1, w2, wf, whp, g1, b1, g2, b2, bf, bhp)


def _sc_scatter_body(covered, mean_hbm, pb_hbm, out_hbm, mean_v, idx_v, buf_v):
    nch = covered // _CH
    rem = covered % _CH
    nw = 32
    q, r = nch // nw, nch % nw
    cidx = lax.axis_index("c")
    sidx = lax.axis_index("s")
    w = sidx * 2 + cidx
    start = w * q + jnp.minimum(w, r)
    n_my = q + jnp.where(w < r, 1, 0)
    pltpu.sync_copy(mean_hbm, mean_v)
    lanes = lax.broadcasted_iota(jnp.int32, (_LANES,), 0)

    def _fill_uniform(b):
        for qq in range(4):
            mv = plsc.load_gather(mean_v, [(qq * 16 + lanes) * _SEG + b])
            for l in range(16):
                row = qq * 16 + l
                vs = jnp.broadcast_to(mv[l], (_LANES,))

                @pl.loop(0, _CH // _LANES)
                def _fc(g):
                    buf_v[row, pl.ds(g * _LANES, _LANES)] = vs

    def _fill_mixed(ngroups):
        @pl.loop(0, ngroups)
        def _fg(g):
            pbv = idx_v[pl.ds(g * _LANES, _LANES)]

            @pl.loop(0, 64)
            def _fj(f):
                vals = plsc.load_gather(mean_v, [pbv + f * _SEG])
                buf_v[f, pl.ds(g * _LANES, _LANES)] = vals

    @pl.loop(0, n_my, init_carry=jnp.int32(-1))
    def _chunk(k, cur_b):
        c = start + k
        base = c * _CH
        pltpu.sync_copy(pb_hbm.at[pl.ds(base, _CH)], idx_v)
        b0 = idx_v[pl.ds(0, _LANES)][0]
        b1 = idx_v[pl.ds(_CH - _LANES, _LANES)][_LANES - 1]
        uniform = b0 == b1

        @pl.when(uniform & (b0 != cur_b))
        def _():
            _fill_uniform(b0)

        @pl.when(jnp.logical_not(uniform))
        def _():
            _fill_mixed(_CH // _LANES)

        pltpu.sync_copy(buf_v, out_hbm.at[:, pl.ds(base, _CH)])
        return jnp.where(uniform, b0, jnp.int32(-1))

    if rem:
        @pl.when(w == nw - 1)
        def _tail():
            base = nch * _CH
            pltpu.sync_copy(pb_hbm.at[pl.ds(base, rem)],
                            idx_v.at[pl.ds(0, rem)])
            b0 = idx_v[pl.ds(0, _LANES)][0]
            b1 = idx_v[pl.ds(rem - _LANES, _LANES)][_LANES - 1]
            uniform = b0 == b1

            @pl.when(uniform & (b0 != _chunk))
            def _():
                _fill_uniform(b0)

            @pl.when(jnp.logical_not(uniform))
            def _():
                _fill_mixed(rem // _LANES)

            pltpu.sync_copy(buf_v.at[:, pl.ds(0, rem)],
                            out_hbm.at[:, pl.ds(base, rem)])


def _run_scatter(mean_flat, pb, npoint, covered):
    mesh = plsc.VectorSubcoreMesh(core_axis_name="c", subcore_axis_name="s")
    return pl.kernel(
        functools.partial(_sc_scatter_body, covered),
        out_type=jax.ShapeDtypeStruct((64, npoint), jnp.float32),
        mesh=mesh,
        scratch_types=[
            pltpu.VMEM((_SEG * 64,), jnp.float32),
            pltpu.VMEM((_CH,), jnp.int32),
            pltpu.VMEM((64, _CH), jnp.float32),
        ],
        compiler_params=pltpu.CompilerParams(
            needs_layout_passes=False, use_tc_tiling_on_sc=True),
    )(mean_flat, pb)


def kernel(voxel_feats, voxel_coors, pts_coors, voxel_shape, stride,
           W1, g1, b1, W2, g2, b2, Wh, bh, Wf, bf):
    nvox = voxel_feats.shape[0]
    npoint = pts_coors.shape[0]
    xt = voxel_feats.T
    vbt = voxel_coors.T
    whp = jnp.zeros((_SEG, 64), jnp.float32).at[:3, :].set(Wh)
    bhp = jnp.zeros((_SEG, 1), jnp.float32).at[:3, 0].set(bh)
    c1 = lambda v: v.reshape(64, 1)
    mean_t, _cnt, loss_o, pb2 = _run_mlp_seg(
        xt, vbt, pts_coors.T, W1, W2, Wf, whp,
        c1(g1), c1(b1), c1(g2), c1(b2), c1(bf), bhp)
    mean_flat = mean_t.reshape(-1)
    pb = pb2.reshape(npoint)
    covered = (npoint // 128) * 128
    point_t = _run_scatter(mean_flat, pb, npoint, covered)
    if covered < npoint:
        tail = jnp.take(mean_t, pb[covered:], axis=1)
        point_t = lax.dynamic_update_slice(point_t, tail, (0, covered))
    point_feats = point_t.T
    offset_reg_loss = loss_o[0, 0]
    return (point_feats, offset_reg_loss)


# speedup vs baseline: 1.1916x; 1.1916x over previous
import functools

import jax
import jax.numpy as jnp
from jax import lax
from jax.experimental import pallas as pl
from jax.experimental.pallas import tpu as pltpu
from jax.experimental.pallas import tpu_sc as plsc

_EPS = 1e-5
_OFFSET_RANGE = 2.0
_SEG = 8
_TC_T = 8192
_CH = 1024
_LANES = 16


def _mlp_seg_kernel(nblk, nvox, x_ref, vb_ref, pts_ref, w1_ref, w2_ref, wf_ref,
                    wh_ref, g1_ref, b1_ref, g2_ref, b2_ref, bf_ref, bh_ref,
                    mean_ref, cnt_ref, loss_ref, pb_ref):
    i = pl.program_id(0)
    pb_ref[...] = pts_ref[0, :]
    inv_std = 1.0 / jnp.sqrt(1.0 + _EPS)
    x = x_ref[...]
    h = jnp.dot(w1_ref[...], x, preferred_element_type=jnp.float32)
    h = jnp.maximum(h * (g1_ref[...] * inv_std) + b1_ref[...], 0.0)
    h = jnp.dot(w2_ref[...], h, preferred_element_type=jnp.float32)
    h = jnp.maximum(h * (g2_ref[...] * inv_std) + b2_ref[...], 0.0)
    col = lax.broadcasted_iota(jnp.int32, (1, _TC_T), 1) + i * _TC_T
    valid = col < nvox
    off = jnp.tanh(jnp.dot(wh_ref[...], h, preferred_element_type=jnp.float32)
                   + bh_ref[...]) * _OFFSET_RANGE
    loss_part = jnp.sum(jnp.abs(jnp.where(valid, off, 0.0)))
    aligned = jnp.dot(wf_ref[...], h, preferred_element_type=jnp.float32) + bf_ref[...]
    aligned = jnp.where(valid, aligned, 0.0)
    vb = vb_ref[0:1, :]
    seg_iota = lax.broadcasted_iota(jnp.int32, (_SEG, _TC_T), 0)
    oh = ((vb == seg_iota) & valid).astype(jnp.float32)
    dnums = (((1,), (1,)), ((), ()))
    part_sums = lax.dot_general(aligned, oh, dnums,
                                preferred_element_type=jnp.float32)
    part_cnt = lax.dot_general(jnp.ones((64, _TC_T), jnp.float32), oh, dnums,
                               preferred_element_type=jnp.float32)

    @pl.when(i == 0)
    def _init():
        mean_ref[...] = jnp.zeros_like(mean_ref)
        cnt_ref[...] = jnp.zeros_like(cnt_ref)
        loss_ref[...] = jnp.zeros_like(loss_ref)

    mean_ref[...] += part_sums
    cnt_ref[...] += part_cnt
    loss_ref[...] += loss_part

    @pl.when(i == nblk - 1)
    def _fin():
        c = cnt_ref[...]
        s = mean_ref[...]
        mean_ref[...] = jnp.where(c > 0.0, s / jnp.maximum(c, 1.0), 0.0)
        loss_ref[...] = loss_ref[...] / (nvox * 3.0)


def _run_mlp_seg(xt, vbt, ptst, w1, w2, wf, whp, g1, b1, g2, b2, bf, bhp):
    nvox = xt.shape[1]
    npoint = ptst.shape[1]
    nblk = (nvox + _TC_T - 1) // _TC_T
    pt_t = (npoint // nblk + 1023) // 1024 * 1024
    col = lambda: pl.BlockSpec((64, 1), lambda i: (0, 0))
    wspec = lambda shp: pl.BlockSpec(shp, lambda i: (0, 0))
    return pl.pallas_call(
        functools.partial(_mlp_seg_kernel, nblk, nvox),
        grid=(nblk,),
        in_specs=[
            pl.BlockSpec((64, _TC_T), lambda i: (0, i)),
            pl.BlockSpec((4, _TC_T), lambda i: (0, i)),
            pl.BlockSpec((3, pt_t), lambda i: (0, i)),
            wspec((64, 64)), wspec((64, 64)), wspec((64, 64)), wspec((_SEG, 64)),
            col(), col(), col(), col(), col(),
            wspec((_SEG, 1)),
        ],
        out_specs=[
            pl.BlockSpec((64, _SEG), lambda i: (0, 0)),
            pl.BlockSpec((64, _SEG), lambda i: (0, 0)),
            pl.BlockSpec((_SEG, 128), lambda i: (0, 0)),
            pl.BlockSpec((pt_t,), lambda i: (i,)),
        ],
        out_shape=[
            jax.ShapeDtypeStruct((64, _SEG), jnp.float32),
            jax.ShapeDtypeStruct((64, _SEG), jnp.float32),
            jax.ShapeDtypeStruct((_SEG, 128), jnp.float32),
            jax.ShapeDtypeStruct((npoint,), jnp.int32),
        ],
    )(xt, vbt, ptst, w1, w2, wf, whp, g1, b1, g2, b2, bf, bhp)


def _sc_scatter_body(covered, mean_hbm, pb_hbm, out_hbm, mean_v, idx_v, buf_v):
    nch = covered // _CH
    rem = covered % _CH
    nw = 32
    q, r = nch // nw, nch % nw
    cidx = lax.axis_index("c")
    sidx = lax.axis_index("s")
    w = sidx * 2 + cidx
    start = w * q + jnp.minimum(w, r)
    n_my = q + jnp.where(w < r, 1, 0)
    pltpu.sync_copy(mean_hbm, mean_v)
    lanes = lax.broadcasted_iota(jnp.int32, (_LANES,), 0)

    def _fill_uniform(b):
        for qq in range(4):
            mv = plsc.load_gather(mean_v, [(qq * 16 + lanes) * _SEG + b])
            for l in range(16):
                row = qq * 16 + l
                vs = jnp.broadcast_to(mv[l], (_LANES,))

                @pl.loop(0, _CH // _LANES)
                def _fc(g):
                    buf_v[row, pl.ds(g * _LANES, _LANES)] = vs

    def _fill_mixed(ngroups):
        @pl.loop(0, ngroups)
        def _fg(g):
            pbv = idx_v[pl.ds(g * _LANES, _LANES)]

            @pl.loop(0, 64)
            def _fj(f):
                vals = plsc.load_gather(mean_v, [pbv + f * _SEG])
                buf_v[f, pl.ds(g * _LANES, _LANES)] = vals

    @pl.loop(0, n_my, init_carry=jnp.int32(-1))
    def _chunk(k, cur_b):
        c = start + k
        base = c * _CH
        pltpu.sync_copy(pb_hbm.at[pl.ds(base, _CH)], idx_v)
        b0 = idx_v[pl.ds(0, _LANES)][0]
        b1 = idx_v[pl.ds(_CH - _LANES, _LANES)][_LANES - 1]
        uniform = b0 == b1

        @pl.when(uniform & (b0 != cur_b))
        def _():
            _fill_uniform(b0)

        @pl.when(jnp.logical_not(uniform))
        def _():
            _fill_mixed(_CH // _LANES)

        pltpu.sync_copy(buf_v, out_hbm.at[:, pl.ds(base, _CH)])
        return jnp.where(uniform, b0, jnp.int32(-1))

    if rem:
        @pl.when(w == nw - 1)
        def _tail():
            base = nch * _CH
            pltpu.sync_copy(pb_hbm.at[pl.ds(base, rem)],
                            idx_v.at[pl.ds(0, rem)])
            b0 = idx_v[pl.ds(0, _LANES)][0]
            b1 = idx_v[pl.ds(rem - _LANES, _LANES)][_LANES - 1]
            uniform = b0 == b1

            @pl.when(uniform & (b0 != _chunk))
            def _():
                _fill_uniform(b0)

            @pl.when(jnp.logical_not(uniform))
            def _():
                _fill_mixed(rem // _LANES)

            pltpu.sync_copy(buf_v.at[:, pl.ds(0, rem)],
                            out_hbm.at[:, pl.ds(base, rem)])


def _run_scatter(mean_flat, pb, npoint, covered):
    mesh = plsc.VectorSubcoreMesh(core_axis_name="c", subcore_axis_name="s")
    return pl.kernel(
        functools.partial(_sc_scatter_body, covered),
        out_type=jax.ShapeDtypeStruct((64, npoint), jnp.float32),
        mesh=mesh,
        scratch_types=[
            pltpu.VMEM((_SEG * 64,), jnp.float32),
            pltpu.VMEM((_CH,), jnp.int32),
            pltpu.VMEM((64, _CH), jnp.float32),
        ],
        compiler_params=pltpu.CompilerParams(
            needs_layout_passes=False, use_tc_tiling_on_sc=True),
    )(mean_flat, pb)


def kernel(voxel_feats, voxel_coors, pts_coors, voxel_shape, stride,
           W1, g1, b1, W2, g2, b2, Wh, bh, Wf, bf):
    nvox = voxel_feats.shape[0]
    npoint = pts_coors.shape[0]
    xt = voxel_feats.T
    vbt = voxel_coors.T
    whp = jnp.zeros((_SEG, 64), jnp.float32).at[:3, :].set(Wh)
    bhp = jnp.zeros((_SEG, 1), jnp.float32).at[:3, 0].set(bh)
    c1 = lambda v: v.reshape(64, 1)
    mean_t, _cnt, loss_o, pb2 = _run_mlp_seg(
        xt, vbt, pts_coors.T, W1, W2, Wf, whp,
        c1(g1), c1(b1), c1(g2), c1(b2), c1(bf), bhp)
    mean_flat = mean_t.reshape(-1)
    pb = pb2
    covered = (npoint // 128) * 128
    point_t = _run_scatter(mean_flat, pb, npoint, covered)
    if covered < npoint:
        tail = jnp.take(mean_t, pb[covered:], axis=1)
        point_t = lax.dynamic_update_slice(point_t, tail, (0, covered))
    point_feats = point_t.T
    offset_reg_loss = loss_o[0, 0]
    return (point_feats, offset_reg_loss)
